# Initial kernel scaffold; baseline (speedup 1.0000x reference)
import jax
import jax.numpy as jnp
from jax.experimental import pallas as pl


def kernel(x, edge_index, h, W, V, alpha):
    raise NotImplementedError("write your pallas kernel here")


# speedup vs baseline: 4.8671x; 4.8671x over previous
import functools

import jax
import jax.numpy as jnp
from jax import lax
from jax.experimental import pallas as pl
from jax.experimental.pallas import tpu as pltpu
from jax.experimental.pallas import tpu_sc as plsc

N_NODES = 10000
F_DIM = 128

NC = 2
NS = 16
NW = NC * NS

CHUNK = 128
ROWS_PER_TILE = 640
WB_ROWS = 64
ACC_ROWS = NS * ROWS_PER_TILE


def _sc_body(src_hbm, dst_hbm, h_hbm, out_hbm,
             sidx, didx, rows, zbuf, wbuf, acc, gsem):
    c = lax.axis_index("c")
    s = lax.axis_index("s")
    wid = s * NC + c
    n_chunks = src_hbm.shape[0] // NW // CHUNK
    base = wid * (n_chunks * CHUNK)

    for i in range(16):
        for j in range(8):
            zbuf[i, pl.ds(j * 16, 16)] = jnp.zeros((16,), jnp.float32)
    r0 = s * ROWS_PER_TILE

    def zero_step(k, carry):
        pltpu.sync_copy(zbuf, acc.at[pl.ds(r0 + k * 16, 16)])
        return carry

    lax.fori_loop(0, ROWS_PER_TILE // 16, zero_step, 0)
    plsc.subcore_barrier()

    def edge_step(j, carry):
        off = base + j * CHUNK
        pltpu.sync_copy(src_hbm.at[pl.ds(off, CHUNK)], sidx)
        pltpu.sync_copy(dst_hbm.at[pl.ds(off, CHUNK)], didx)
        pltpu.async_copy(h_hbm.at[sidx], rows, gsem).wait()
        pltpu.sync_copy(rows, acc.at[didx], add=True)
        return carry

    lax.fori_loop(0, n_chunks, edge_step, 0)
    plsc.subcore_barrier()

    def wb_step(k, carry):
        rr = r0 + k * WB_ROWS
        pltpu.sync_copy(acc.at[pl.ds(rr, WB_ROWS)], wbuf)
        pltpu.sync_copy(wbuf, out_hbm.at[c, pl.ds(rr, WB_ROWS)])
        return carry

    lax.fori_loop(0, ROWS_PER_TILE // WB_ROWS, wb_step, 0)


def _sc_segment_sum(src, dst, h):
    mesh = plsc.VectorSubcoreMesh(core_axis_name="c", subcore_axis_name="s")
    fn = pl.kernel(
        _sc_body,
        out_type=jax.ShapeDtypeStruct((NC, ACC_ROWS, F_DIM), jnp.float32),
        mesh=mesh,
        scratch_types=[
            pltpu.VMEM((CHUNK,), jnp.int32),
            pltpu.VMEM((CHUNK,), jnp.int32),
            pltpu.VMEM((CHUNK, F_DIM), jnp.float32),
            pltpu.VMEM((16, F_DIM), jnp.float32),
            pltpu.VMEM((WB_ROWS, F_DIM), jnp.float32),
            pltpu.VMEM_SHARED((ACC_ROWS, F_DIM), jnp.float32),
            pltpu.SemaphoreType.DMA,
        ],
    )
    return fn(src, dst, h)


def _tc_fuse_body(x_ref, w_ref, v_ref, p0_ref, p1_ref, o_ref):
    agg = p0_ref[...] + p1_ref[...]
    o_ref[...] = jnp.maximum(
        jnp.dot(x_ref[...], w_ref[...], preferred_element_type=jnp.float32)
        + jnp.dot(agg, v_ref[...], preferred_element_type=jnp.float32),
        0.0,
    )


def _tc_fuse(x, W, V, p0, p1):
    blk = 400
    grid = (N_NODES // blk,)
    return pl.pallas_call(
        _tc_fuse_body,
        grid=grid,
        in_specs=[
            pl.BlockSpec((blk, F_DIM), lambda i: (i, 0)),
            pl.BlockSpec((F_DIM, F_DIM), lambda i: (0, 0)),
            pl.BlockSpec((F_DIM, F_DIM), lambda i: (0, 0)),
            pl.BlockSpec((blk, F_DIM), lambda i: (i, 0)),
            pl.BlockSpec((blk, F_DIM), lambda i: (i, 0)),
        ],
        out_specs=pl.BlockSpec((blk, F_DIM), lambda i: (i, 0)),
        out_shape=jax.ShapeDtypeStruct((N_NODES, F_DIM), jnp.float32),
    )(x, W, V, p0, p1)


def kernel(x, edge_index, h, W, V, alpha):
    ei = edge_index.astype(jnp.int32)
    src = ei[:, 0, :].reshape(-1)
    dst = ei[:, 1, :].reshape(-1)
    total = src.shape[0]
    per_w = -(-total // (NW * CHUNK)) * CHUNK
    pad = NW * per_w - total
    if pad:
        src = jnp.concatenate([src, jnp.zeros((pad,), jnp.int32)])
        dst = jnp.concatenate([dst, jnp.full((pad,), N_NODES, jnp.int32)])
    partials = _sc_segment_sum(src, dst, h)
    return _tc_fuse(x, W, V, partials[0, :N_NODES], partials[1, :N_NODES])


# speedup vs baseline: 6.0721x; 1.2476x over previous
import functools

import jax
import jax.numpy as jnp
from jax import lax
from jax.experimental import pallas as pl
from jax.experimental.pallas import tpu as pltpu
from jax.experimental.pallas import tpu_sc as plsc

N_NODES = 10000
F_DIM = 128

NC = 2
NS = 16
NW = NC * NS

CHUNK = 128
ROWS_PER_TILE = 640
WB_ROWS = 64
ACC_ROWS = NS * ROWS_PER_TILE


def _sc_body(src_hbm, dst_hbm, h_hbm, out_hbm,
             sidx0, sidx1, didx0, didx1, rows0, rows1,
             zbuf, wbuf, acc, gsem0, gsem1):
    c = lax.axis_index("c")
    s = lax.axis_index("s")
    wid = s * NC + c
    n_chunks = src_hbm.shape[0] // NW // CHUNK
    base = wid * (n_chunks * CHUNK)
    sidx = (sidx0, sidx1)
    didx = (didx0, didx1)
    rows = (rows0, rows1)
    gsem = (gsem0, gsem1)

    for i in range(16):
        for j in range(8):
            zbuf[i, pl.ds(j * 16, 16)] = jnp.zeros((16,), jnp.float32)
    r0 = s * ROWS_PER_TILE

    def zero_step(k, carry):
        pltpu.sync_copy(zbuf, acc.at[pl.ds(r0 + k * 16, 16)])
        return carry

    lax.fori_loop(0, ROWS_PER_TILE // 16, zero_step, 0)
    plsc.subcore_barrier()

    def load_idx(b, j):
        off = base + j * CHUNK
        pltpu.sync_copy(src_hbm.at[pl.ds(off, CHUNK)], sidx[b])
        pltpu.sync_copy(dst_hbm.at[pl.ds(off, CHUNK)], didx[b])

    def issue_gather(b):
        pltpu.async_copy(h_hbm.at[sidx[b]], rows[b], gsem[b])

    def wait_gather(b):
        pltpu.make_async_copy(h_hbm.at[sidx[b]], rows[b], gsem[b]).wait()

    def scatter(b):
        pltpu.sync_copy(rows[b], acc.at[didx[b]], add=True)

    for b in range(2):
        load_idx(b, b)
        issue_gather(b)

    def edge_step(k, carry):
        for b in range(2):
            j = 2 * k + b
            wait_gather(b)
            scatter(b)
            load_idx(b, j + 2)
            issue_gather(b)
        return carry

    lax.fori_loop(0, n_chunks // 2 - 1, edge_step, 0)
    for b in range(2):
        wait_gather(b)
        scatter(b)
    plsc.subcore_barrier()

    def wb_step(k, carry):
        rr = r0 + k * WB_ROWS
        pltpu.sync_copy(acc.at[pl.ds(rr, WB_ROWS)], wbuf)
        pltpu.sync_copy(wbuf, out_hbm.at[c, pl.ds(rr, WB_ROWS)])
        return carry

    lax.fori_loop(0, ROWS_PER_TILE // WB_ROWS, wb_step, 0)


def _sc_segment_sum(src, dst, h):
    mesh = plsc.VectorSubcoreMesh(core_axis_name="c", subcore_axis_name="s")
    fn = pl.kernel(
        _sc_body,
        out_type=jax.ShapeDtypeStruct((NC, ACC_ROWS, F_DIM), jnp.float32),
        mesh=mesh,
        scratch_types=[
            pltpu.VMEM((CHUNK,), jnp.int32),
            pltpu.VMEM((CHUNK,), jnp.int32),
            pltpu.VMEM((CHUNK,), jnp.int32),
            pltpu.VMEM((CHUNK,), jnp.int32),
            pltpu.VMEM((CHUNK, F_DIM), jnp.float32),
            pltpu.VMEM((CHUNK, F_DIM), jnp.float32),
            pltpu.VMEM((16, F_DIM), jnp.float32),
            pltpu.VMEM((WB_ROWS, F_DIM), jnp.float32),
            pltpu.VMEM_SHARED((ACC_ROWS, F_DIM), jnp.float32),
            pltpu.SemaphoreType.DMA,
            pltpu.SemaphoreType.DMA,
        ],
    )
    return fn(src, dst, h)


def _tc_fuse_body(x_ref, w_ref, v_ref, p0_ref, p1_ref, o_ref):
    agg = p0_ref[...] + p1_ref[...]
    o_ref[...] = jnp.maximum(
        jnp.dot(x_ref[...], w_ref[...], preferred_element_type=jnp.float32)
        + jnp.dot(agg, v_ref[...], preferred_element_type=jnp.float32),
        0.0,
    )


def _tc_fuse(x, W, V, p0, p1):
    blk = 400
    grid = (N_NODES // blk,)
    return pl.pallas_call(
        _tc_fuse_body,
        grid=grid,
        in_specs=[
            pl.BlockSpec((blk, F_DIM), lambda i: (i, 0)),
            pl.BlockSpec((F_DIM, F_DIM), lambda i: (0, 0)),
            pl.BlockSpec((F_DIM, F_DIM), lambda i: (0, 0)),
            pl.BlockSpec((blk, F_DIM), lambda i: (i, 0)),
            pl.BlockSpec((blk, F_DIM), lambda i: (i, 0)),
        ],
        out_specs=pl.BlockSpec((blk, F_DIM), lambda i: (i, 0)),
        out_shape=jax.ShapeDtypeStruct((N_NODES, F_DIM), jnp.float32),
    )(x, W, V, p0, p1)


def kernel(x, edge_index, h, W, V, alpha):
    ei = edge_index.astype(jnp.int32)
    src = ei[:, 0, :].reshape(-1)
    dst = ei[:, 1, :].reshape(-1)
    total = src.shape[0]
    per_w = -(-total // (NW * 2 * CHUNK)) * 2 * CHUNK
    pad = NW * per_w - total
    if pad:
        src = jnp.concatenate([src, jnp.zeros((pad,), jnp.int32)])
        dst = jnp.concatenate([dst, jnp.full((pad,), N_NODES, jnp.int32)])
    partials = _sc_segment_sum(src, dst, h)
    return _tc_fuse(x, W, V, partials[0, :N_NODES], partials[1, :N_NODES])
